# baseline (speedup 1.0000x reference)
import functools

import jax
import jax.numpy as jnp
from jax import lax
from jax.experimental import pallas as pl
from jax.experimental.pallas import tpu as pltpu

B, D, M = 256, 256, 100000
TEMP = 0.05
EPS = 0.1
MARGIN = 0.1
MB = 4000
NBLK = M // MB
NEG = -1e9


def _body(tcol_ref, x_in_ref, feat_ref, sf_ref, lab_ref, out_ce_ref,
          out_l2_ref, s_x, s_se, s_fs, s_tl, s_mp, s_mn, s_pl, s_hp):
    p = pl.program_id(0)
    m = pl.program_id(1)

    @pl.when((p == 0) & (m == 0))
    def _init():
        x = x_in_ref[...]
        xn = x * lax.rsqrt(jnp.sum(x * x, axis=1, keepdims=True))
        s_x[...] = xn.astype(jnp.bfloat16)
        s_se[...] = jnp.zeros((B, 1), jnp.float32)
        s_fs[...] = jnp.zeros((1, D), jnp.float32)
        s_tl[...] = jnp.zeros((B, 1), jnp.float32)
        s_mp[...] = jnp.full((B, 1), NEG, jnp.float32)
        s_mn[...] = jnp.full((B, 1), NEG, jnp.float32)

    x = s_x[...]
    dn = (((1,), (1,)), ((), ()))
    sim = lax.dot_general(x, sf_ref[...].astype(jnp.bfloat16), dn,
                          preferred_element_type=jnp.float32
                          ).astype(jnp.bfloat16)
    lab = lab_ref[0]
    tcol = tcol_ref[...]
    posm = lab == tcol

    @pl.when(p == 0)
    def _ph0():
        f = feat_ref[...].astype(jnp.bfloat16)
        r = lax.dot_general(x, f, dn, preferred_element_type=jnp.float32)
        C = 28.853900817779268
        s_se[...] += jnp.sum(jnp.exp2(r * C), axis=1, keepdims=True)
        ones = jnp.ones((1, MB), jnp.bfloat16)
        s_fs[...] += lax.dot_general(ones, f, (((1,), (0,)), ((), ())),
                                     preferred_element_type=jnp.float32)
        col = m * MB + lax.broadcasted_iota(jnp.int32, (1, MB), 1)
        s_tl[...] += jnp.sum(jnp.where(col == tcol, r, 0.0), axis=1,
                             keepdims=True)
        s_mp[...] = jnp.maximum(
            s_mp[...],
            jnp.max(jnp.where(posm, sim, NEG), axis=1,
                    keepdims=True).astype(jnp.float32))
        s_mn[...] = jnp.maximum(
            s_mn[...],
            jnp.max(jnp.where(posm, NEG, sim), axis=1,
                    keepdims=True).astype(jnp.float32))

    @pl.when((p == 1) & (m == 0))
    def _mid():
        s_hp[...] = jnp.where(s_mp[...] > -1e8, 1.0, 0.0)
        s_mn[...] = s_mn[...] + MARGIN
        s_mp[...] = jnp.maximum(0.6, s_mp[...]) - MARGIN
        s_pl[...] = jnp.zeros((B, 1), jnp.float32)

    @pl.when(p == 1)
    def _ph1():
        tb = s_mn[...].astype(jnp.bfloat16)
        hb = s_mp[...].astype(jnp.bfloat16)
        val = jnp.where(posm & (sim < tb), 1.0 - sim,
                        jnp.where(posm | (sim <= hb), 0.0, sim))
        s_pl[...] += jnp.sum(val, axis=1, keepdims=True).astype(jnp.float32)

    @pl.when((p == 1) & (m == NBLK - 1))
    def _fin():
        C = 28.853900817779268
        lse = (20.0 - C * 0.6931471805599453) + jnp.log(s_se[...])
        so = jnp.sum(x.astype(jnp.float32) * s_fs[...], axis=1,
                     keepdims=True) * (1.0 / TEMP)
        ce = ((1.0 - EPS) * (lse - 20.0 * s_tl[...])
              + (EPS / M) * (M * lse - so))
        out_ce_ref[...] = jnp.sum(ce, keepdims=True).reshape(1, 1) / B
        li = jnp.where(s_hp[...] > 0, s_pl[...], 0.0)
        out_l2_ref[...] = jnp.sum(li, keepdims=True).reshape(1, 1) / B


@functools.partial(jax.jit, static_argnames=("interpret",))
def _run(inputs, targets, features, sample_features, sample_labels,
         interpret=False):
    tcol = targets.reshape(B, 1)
    lab3 = sample_labels.reshape(NBLK, 1, MB)
    f32 = jnp.float32
    out_ce, out_l2 = pl.pallas_call(
        _body,
        grid=(2, NBLK),
        in_specs=[
            pl.BlockSpec((B, 1), lambda p, m: (0, 0)),
            pl.BlockSpec((B, D), lambda p, m: (0, 0)),
            pl.BlockSpec((MB, D), lambda p, m: (m * (1 - p), 0)),
            pl.BlockSpec((MB, D), lambda p, m: (m, 0)),
            pl.BlockSpec((1, 1, MB), lambda p, m: (m, 0, 0)),
        ],
        out_specs=[
            pl.BlockSpec((1, 1), lambda p, m: (0, 0)),
            pl.BlockSpec((1, 1), lambda p, m: (0, 0)),
        ],
        out_shape=[
            jax.ShapeDtypeStruct((1, 1), f32),
            jax.ShapeDtypeStruct((1, 1), f32),
        ],
        scratch_shapes=[
            pltpu.VMEM((B, D), jnp.bfloat16),
            pltpu.VMEM((B, 1), f32), pltpu.VMEM((1, D), f32),
            pltpu.VMEM((B, 1), f32), pltpu.VMEM((B, 1), f32),
            pltpu.VMEM((B, 1), f32), pltpu.VMEM((B, 1), f32),
            pltpu.VMEM((B, 1), f32),
        ],
        interpret=interpret,
    )(tcol, inputs, features, sample_features, lab3)
    return out_ce[0, 0], out_l2[0, 0]


def kernel(inputs, targets, features, sample_features, sample_labels):
    return _run(inputs, targets, features, sample_features, sample_labels)


# speedup vs baseline: 1.1442x; 1.1442x over previous
import functools

import jax
import jax.numpy as jnp
from jax import lax
from jax.experimental import pallas as pl
from jax.experimental.pallas import tpu as pltpu
from jax.experimental.pallas import tpu_sc as plsc

B, D, M = 256, 256, 100000
TEMP = 0.05
EPS = 0.1
MARGIN = 0.1
MB = 4000
NBLK = M // MB
NEG = -1e9


def _sc_gather(features, targets):
    info = plsc.get_sparse_core_info()
    nc, ns = info.num_cores, info.num_subcores
    bpw = B // (nc * ns)
    mesh = plsc.VectorSubcoreMesh(core_axis_name="c", subcore_axis_name="s")

    @functools.partial(
        pl.kernel, mesh=mesh,
        out_type=jax.ShapeDtypeStruct((B, D), jnp.float32),
        scratch_types=[
            pltpu.VMEM((bpw,), jnp.int32),
            pltpu.VMEM((bpw, D), jnp.float32),
            pltpu.SemaphoreType.DMA,
        ],
    )
    def gk(table_hbm, idx_hbm, out_hbm, idx_v, rows_v, sem):
        wid = lax.axis_index("s") * nc + lax.axis_index("c")
        base = wid * bpw
        pltpu.sync_copy(idx_hbm.at[pl.ds(base, bpw)], idx_v)
        pltpu.async_copy(table_hbm.at[idx_v], rows_v, sem).wait()
        pltpu.sync_copy(rows_v, out_hbm.at[pl.ds(base, bpw)])

    return gk(features, targets)


def _body(tcol_ref, x_in_ref, feat_ref, sf_ref, lab_ref, g_ref, out_ce_ref,
          out_l2_ref, s_x, s_se, s_fs, s_mp, s_mn, s_pl, s_hp):
    p = pl.program_id(0)
    m = pl.program_id(1)

    @pl.when((p == 0) & (m == 0))
    def _init():
        x = x_in_ref[...]
        xn = x * lax.rsqrt(jnp.sum(x * x, axis=1, keepdims=True))
        s_x[...] = xn.astype(jnp.bfloat16)
        s_se[...] = jnp.zeros((B, 1), jnp.float32)
        s_fs[...] = jnp.zeros((1, D), jnp.float32)
        s_mp[...] = jnp.full((B, 1), NEG, jnp.float32)
        s_mn[...] = jnp.full((B, 1), NEG, jnp.float32)

    x = s_x[...]
    dn = (((1,), (1,)), ((), ()))
    sim = lax.dot_general(x, sf_ref[...].astype(jnp.bfloat16), dn,
                          preferred_element_type=jnp.float32)
    lab = lab_ref[0]
    tcol = tcol_ref[...]
    posm = lab == tcol

    @pl.when(p == 0)
    def _ph0():
        f = feat_ref[...].astype(jnp.bfloat16)
        r = lax.dot_general(x, f, dn, preferred_element_type=jnp.float32)
        C = 28.853900817779268
        s_se[...] += jnp.sum(jnp.exp2(r * C), axis=1, keepdims=True)
        ones = jnp.ones((1, MB), jnp.bfloat16)
        s_fs[...] += lax.dot_general(ones, f, (((1,), (0,)), ((), ())),
                                     preferred_element_type=jnp.float32)
        s_mp[...] = jnp.maximum(
            s_mp[...], jnp.max(jnp.where(posm, sim, NEG), axis=1, keepdims=True))
        s_mn[...] = jnp.maximum(
            s_mn[...], jnp.max(jnp.where(posm, NEG, sim), axis=1, keepdims=True))

    @pl.when((p == 1) & (m == 0))
    def _mid():
        s_hp[...] = jnp.where(s_mp[...] > -1e8, 1.0, 0.0)
        s_mn[...] = s_mn[...] + MARGIN
        s_mp[...] = jnp.maximum(0.6, s_mp[...]) - MARGIN
        s_pl[...] = jnp.zeros((B, 1), jnp.float32)

    @pl.when(p == 1)
    def _ph1():
        val = jnp.where(posm & (sim < s_mn[...]), 1.0 - sim,
                        jnp.where(posm | (sim <= s_mp[...]), 0.0, sim))
        s_pl[...] += jnp.sum(val, axis=1, keepdims=True)

    @pl.when((p == 1) & (m == NBLK - 1))
    def _fin():
        lse = jnp.log(s_se[...])
        xi = x_in_ref[...]
        xn = xi * lax.rsqrt(jnp.sum(xi * xi, axis=1, keepdims=True))
        tl = jnp.sum(xn * g_ref[...], axis=1, keepdims=True) * (1.0 / TEMP)
        so = jnp.sum(xn * s_fs[...], axis=1, keepdims=True) * (1.0 / TEMP)
        ce = ((1.0 - EPS) * (lse - tl) + (EPS / M) * (M * lse - so))
        out_ce_ref[...] = jnp.sum(ce, keepdims=True).reshape(1, 1) / B
        li = jnp.where(s_hp[...] > 0, s_pl[...], 0.0)
        out_l2_ref[...] = jnp.sum(li, keepdims=True).reshape(1, 1) / B


@jax.jit
def _run(inputs, targets, features, sample_features, sample_labels):
    tcol = targets.reshape(B, 1)
    lab3 = sample_labels.reshape(NBLK, 1, MB)
    g = _sc_gather(features, targets)
    f32 = jnp.float32
    out_ce, out_l2 = pl.pallas_call(
        _body,
        grid=(2, NBLK),
        in_specs=[
            pl.BlockSpec((B, 1), lambda p, m: (0, 0)),
            pl.BlockSpec((B, D), lambda p, m: (0, 0)),
            pl.BlockSpec((MB, D), lambda p, m: (m * (1 - p), 0)),
            pl.BlockSpec((MB, D), lambda p, m: (m, 0)),
            pl.BlockSpec((1, 1, MB), lambda p, m: (m, 0, 0)),
            pl.BlockSpec((B, D), lambda p, m: (0, 0)),
        ],
        out_specs=[
            pl.BlockSpec((1, 1), lambda p, m: (0, 0)),
            pl.BlockSpec((1, 1), lambda p, m: (0, 0)),
        ],
        out_shape=[
            jax.ShapeDtypeStruct((1, 1), f32),
            jax.ShapeDtypeStruct((1, 1), f32),
        ],
        scratch_shapes=[
            pltpu.VMEM((B, D), jnp.bfloat16),
            pltpu.VMEM((B, 1), f32), pltpu.VMEM((1, D), f32),
            pltpu.VMEM((B, 1), f32), pltpu.VMEM((B, 1), f32),
            pltpu.VMEM((B, 1), f32), pltpu.VMEM((B, 1), f32),
        ],
    )(tcol, inputs, features, sample_features, lab3, g)
    return out_ce[0, 0], out_l2[0, 0]


def kernel(inputs, targets, features, sample_features, sample_labels):
    return _run(inputs, targets, features, sample_features, sample_labels)


# speedup vs baseline: 1.1495x; 1.0047x over previous
import functools

import jax
import jax.numpy as jnp
from jax import lax
from jax.experimental import pallas as pl
from jax.experimental.pallas import tpu as pltpu
from jax.experimental.pallas import tpu_sc as plsc

B, D, M = 256, 256, 100000
TEMP = 0.05
EPS = 0.1
MARGIN = 0.1
MB = 4000
NBLK = M // MB
NEG = -1e9


def _sc_gather(features, targets):
    info = plsc.get_sparse_core_info()
    nc, ns = info.num_cores, info.num_subcores
    bpw = B // (nc * ns)
    mesh = plsc.VectorSubcoreMesh(core_axis_name="c", subcore_axis_name="s")

    @functools.partial(
        pl.kernel, mesh=mesh,
        out_type=jax.ShapeDtypeStruct((B, D), jnp.float32),
        scratch_types=[
            pltpu.VMEM((bpw,), jnp.int32),
            pltpu.VMEM((bpw, D), jnp.float32),
            pltpu.SemaphoreType.DMA,
        ],
    )
    def gk(table_hbm, idx_hbm, out_hbm, idx_v, rows_v, sem):
        wid = lax.axis_index("s") * nc + lax.axis_index("c")
        base = wid * bpw
        pltpu.sync_copy(idx_hbm.at[pl.ds(base, bpw)], idx_v)
        pltpu.async_copy(table_hbm.at[idx_v], rows_v, sem).wait()
        pltpu.sync_copy(rows_v, out_hbm.at[pl.ds(base, bpw)])

    return gk(features, targets)


def _body(tcol_ref, x_in_ref, feat_ref, sf_ref, lab_ref, out_cep_ref,
          out_l2_ref, s_x, s_se, s_fs, s_mp, s_mn, s_pl, s_hp):
    p = pl.program_id(0)
    m = pl.program_id(1)

    @pl.when((p == 0) & (m == 0))
    def _init():
        x = x_in_ref[...]
        xn = x * lax.rsqrt(jnp.sum(x * x, axis=1, keepdims=True))
        s_x[...] = xn.astype(jnp.bfloat16)
        s_se[...] = jnp.zeros((B, 1), jnp.float32)
        s_fs[...] = jnp.zeros((1, D), jnp.float32)
        s_mp[...] = jnp.full((B, 1), NEG, jnp.float32)
        s_mn[...] = jnp.full((B, 1), NEG, jnp.float32)

    x = s_x[...]
    dn = (((1,), (1,)), ((), ()))
    sim = lax.dot_general(x, sf_ref[...].astype(jnp.bfloat16), dn,
                          preferred_element_type=jnp.float32)
    lab = lab_ref[0]
    tcol = tcol_ref[...]
    posm = lab == tcol

    @pl.when(p == 0)
    def _ph0():
        f = feat_ref[...].astype(jnp.bfloat16)
        r = lax.dot_general(x, f, dn, preferred_element_type=jnp.float32)
        C = 28.853900817779268
        s_se[...] += jnp.sum(jnp.exp2(r * C), axis=1, keepdims=True)
        ones = jnp.ones((1, MB), jnp.bfloat16)
        s_fs[...] += lax.dot_general(ones, f, (((1,), (0,)), ((), ())),
                                     preferred_element_type=jnp.float32)
        s_mp[...] = jnp.maximum(
            s_mp[...], jnp.max(jnp.where(posm, sim, NEG), axis=1, keepdims=True))
        s_mn[...] = jnp.maximum(
            s_mn[...], jnp.max(jnp.where(posm, NEG, sim), axis=1, keepdims=True))

    @pl.when((p == 1) & (m == 0))
    def _mid():
        s_hp[...] = jnp.where(s_mp[...] > -1e8, 1.0, 0.0)
        s_mn[...] = s_mn[...] + MARGIN
        s_mp[...] = jnp.maximum(0.6, s_mp[...]) - MARGIN
        s_pl[...] = jnp.zeros((B, 1), jnp.float32)

    @pl.when(p == 1)
    def _ph1():
        val = jnp.where(posm & (sim < s_mn[...]), 1.0 - sim,
                        jnp.where(posm | (sim <= s_mp[...]), 0.0, sim))
        s_pl[...] += jnp.sum(val, axis=1, keepdims=True)

    @pl.when((p == 1) & (m == NBLK - 1))
    def _fin():
        lse = jnp.log(s_se[...])
        xi = x_in_ref[...]
        xn = xi * lax.rsqrt(jnp.sum(xi * xi, axis=1, keepdims=True))
        so = jnp.sum(xn * s_fs[...], axis=1, keepdims=True) * (1.0 / TEMP)
        out_cep_ref[...] = ((1.0 - EPS) * lse
                            + (EPS / M) * (M * lse - so))
        li = jnp.where(s_hp[...] > 0, s_pl[...], 0.0)
        out_l2_ref[...] = jnp.sum(li, keepdims=True).reshape(1, 1) / B


def _combine(x_in_ref, g_ref, cep_ref, out_ce_ref):
    xi = x_in_ref[...]
    xn = xi * lax.rsqrt(jnp.sum(xi * xi, axis=1, keepdims=True))
    tl = jnp.sum(xn * g_ref[...], axis=1, keepdims=True) * (1.0 / TEMP)
    ce = cep_ref[...] - (1.0 - EPS) * tl
    out_ce_ref[...] = jnp.sum(ce, keepdims=True).reshape(1, 1) / B


@jax.jit
def _run(inputs, targets, features, sample_features, sample_labels):
    tcol = targets.reshape(B, 1)
    lab3 = sample_labels.reshape(NBLK, 1, MB)
    g = _sc_gather(features, targets)
    f32 = jnp.float32
    out_cep, out_l2 = pl.pallas_call(
        _body,
        grid=(2, NBLK),
        in_specs=[
            pl.BlockSpec((B, 1), lambda p, m: (0, 0)),
            pl.BlockSpec((B, D), lambda p, m: (0, 0)),
            pl.BlockSpec((MB, D), lambda p, m: (m * (1 - p), 0)),
            pl.BlockSpec((MB, D), lambda p, m: (m, 0)),
            pl.BlockSpec((1, 1, MB), lambda p, m: (m, 0, 0)),
        ],
        out_specs=[
            pl.BlockSpec((B, 1), lambda p, m: (0, 0)),
            pl.BlockSpec((1, 1), lambda p, m: (0, 0)),
        ],
        out_shape=[
            jax.ShapeDtypeStruct((B, 1), f32),
            jax.ShapeDtypeStruct((1, 1), f32),
        ],
        scratch_shapes=[
            pltpu.VMEM((B, D), jnp.bfloat16),
            pltpu.VMEM((B, 1), f32), pltpu.VMEM((1, D), f32),
            pltpu.VMEM((B, 1), f32), pltpu.VMEM((B, 1), f32),
            pltpu.VMEM((B, 1), f32), pltpu.VMEM((B, 1), f32),
        ],
    )(tcol, inputs, features, sample_features, lab3)
    out_ce = pl.pallas_call(
        _combine,
        out_shape=jax.ShapeDtypeStruct((1, 1), f32),
    )(inputs, g, out_cep)
    return out_ce[0, 0], out_l2[0, 0]


def kernel(inputs, targets, features, sample_features, sample_labels):
    return _run(inputs, targets, features, sample_features, sample_labels)


# speedup vs baseline: 1.2557x; 1.0924x over previous
import functools

import jax
import jax.numpy as jnp
from jax import lax
from jax.experimental import pallas as pl
from jax.experimental.pallas import tpu as pltpu

B, D, M = 256, 256, 100000
TEMP = 0.05
EPS = 0.1
MARGIN = 0.1
MB = 4000
NBLK = M // MB
NEG = -1e9
LOG2E = 1.4426950408889634
C = 20.0 * LOG2E


def _body(tcol_ref, x_in_ref, feat_ref, sf_ref, lab_ref, out_ce_ref,
          out_l2_ref, s_x, s_xc, s_se, s_fs, s_tl, s_mp, s_mn, s_pl, s_hp):
    p = pl.program_id(0)
    m = pl.program_id(1)

    @pl.when((p == 0) & (m == 0))
    def _init():
        x = x_in_ref[...]
        xn = x * lax.rsqrt(jnp.sum(x * x, axis=1, keepdims=True))
        s_x[...] = xn.astype(jnp.bfloat16)
        s_xc[...] = (xn * C).astype(jnp.bfloat16)
        s_se[...] = jnp.zeros((B, 1), jnp.float32)
        s_fs[...] = jnp.zeros((1, D), jnp.float32)
        s_tl[...] = jnp.zeros((B, 1), jnp.float32)
        s_mp[...] = jnp.full((B, 1), NEG, jnp.float32)
        s_mn[...] = jnp.full((B, 1), NEG, jnp.float32)

    x = s_x[...]
    dn = (((1,), (1,)), ((), ()))
    sim = lax.dot_general(x, sf_ref[...].astype(jnp.bfloat16), dn,
                          preferred_element_type=jnp.float32)
    lab = lab_ref[0]
    tcol = tcol_ref[...]
    posm = lab == tcol

    @pl.when(p == 0)
    def _ph0():
        f = feat_ref[...].astype(jnp.bfloat16)
        rc = lax.dot_general(s_xc[...], f, dn,
                             preferred_element_type=jnp.float32)
        s_se[...] += jnp.sum(jnp.exp2(rc), axis=1, keepdims=True)
        ones = jnp.ones((1, MB), jnp.bfloat16)
        s_fs[...] += lax.dot_general(ones, f, (((1,), (0,)), ((), ())),
                                     preferred_element_type=jnp.float32)
        col = m * MB + lax.broadcasted_iota(jnp.int32, (1, MB), 1)
        s_tl[...] += jnp.sum(jnp.where(col == tcol, rc, 0.0), axis=1,
                             keepdims=True)
        s_mp[...] = jnp.maximum(
            s_mp[...], jnp.max(jnp.where(posm, sim, NEG), axis=1, keepdims=True))
        s_mn[...] = jnp.maximum(
            s_mn[...], jnp.max(jnp.where(posm, NEG, sim), axis=1, keepdims=True))

    @pl.when((p == 1) & (m == 0))
    def _mid():
        s_hp[...] = jnp.where(s_mp[...] > -1e8, 1.0, 0.0)
        s_mn[...] = s_mn[...] + MARGIN
        s_mp[...] = jnp.maximum(0.6, s_mp[...]) - MARGIN
        s_pl[...] = jnp.zeros((B, 1), jnp.float32)

    @pl.when(p == 1)
    def _ph1():
        val = jnp.where(posm & (sim < s_mn[...]), 1.0 - sim,
                        jnp.where(posm | (sim <= s_mp[...]), 0.0, sim))
        s_pl[...] += jnp.sum(val, axis=1, keepdims=True)

    @pl.when((p == 1) & (m == NBLK - 1))
    def _fin():
        lse = jnp.log(s_se[...])
        xi = x_in_ref[...]
        xn = xi * lax.rsqrt(jnp.sum(xi * xi, axis=1, keepdims=True))
        so = jnp.sum(xn * s_fs[...], axis=1, keepdims=True) * (1.0 / TEMP)
        tl = s_tl[...] * (20.0 / C)
        ce = ((1.0 - EPS) * (lse - tl) + (EPS / M) * (M * lse - so))
        out_ce_ref[...] = jnp.sum(ce, keepdims=True).reshape(1, 1) / B
        li = jnp.where(s_hp[...] > 0, s_pl[...], 0.0)
        out_l2_ref[...] = jnp.sum(li, keepdims=True).reshape(1, 1) / B


@functools.partial(jax.jit, static_argnames=("interpret",))
def _run(inputs, targets, features, sample_features, sample_labels,
         interpret=False):
    tcol = targets.reshape(B, 1)
    lab3 = sample_labels.reshape(NBLK, 1, MB)
    f32 = jnp.float32
    out_ce, out_l2 = pl.pallas_call(
        _body,
        grid=(2, NBLK),
        in_specs=[
            pl.BlockSpec((B, 1), lambda p, m: (0, 0)),
            pl.BlockSpec((B, D), lambda p, m: (0, 0)),
            pl.BlockSpec((MB, D), lambda p, m: (m * (1 - p), 0)),
            pl.BlockSpec((MB, D), lambda p, m: (m, 0)),
            pl.BlockSpec((1, 1, MB), lambda p, m: (m, 0, 0)),
        ],
        out_specs=[
            pl.BlockSpec((1, 1), lambda p, m: (0, 0)),
            pl.BlockSpec((1, 1), lambda p, m: (0, 0)),
        ],
        out_shape=[
            jax.ShapeDtypeStruct((1, 1), f32),
            jax.ShapeDtypeStruct((1, 1), f32),
        ],
        scratch_shapes=[
            pltpu.VMEM((B, D), jnp.bfloat16), pltpu.VMEM((B, D), jnp.bfloat16),
            pltpu.VMEM((B, 1), f32), pltpu.VMEM((1, D), f32),
            pltpu.VMEM((B, 1), f32), pltpu.VMEM((B, 1), f32),
            pltpu.VMEM((B, 1), f32), pltpu.VMEM((B, 1), f32),
            pltpu.VMEM((B, 1), f32),
        ],
        interpret=interpret,
    )(tcol, inputs, features, sample_features, lab3)
    return out_ce[0, 0], out_l2[0, 0]


def kernel(inputs, targets, features, sample_features, sample_labels):
    return _run(inputs, targets, features, sample_features, sample_labels)
